# baseline (speedup 1.0000x reference)
import functools

import jax
import jax.numpy as jnp
from jax import lax
from jax.experimental import pallas as pl
from jax.experimental.pallas import tpu as pltpu
from jax.experimental.pallas import tpu_sc as plsc

NC = 2
NS = 16


def _mesh():
    return plsc.VectorSubcoreMesh(core_axis_name="c", subcore_axis_name="s")


def _sc_degree(dst2d, rows, rpw, nblk, bs, ch):
    nch = nblk * bs

    @functools.partial(
        pl.kernel,
        out_type=jax.ShapeDtypeStruct((NC * rows,), jnp.float32),
        mesh=_mesh(),
        scratch_types=[
            pltpu.VMEM((bs, ch), jnp.int32),
            pltpu.VMEM((bs, ch), jnp.int32),
            pltpu.VMEM((ch,), jnp.float32),
            pltpu.VMEM_SHARED((rows,), jnp.float32),
            pltpu.SemaphoreType.DMA,
            pltpu.SemaphoreType.DMA,
            pltpu.SemaphoreType.DMA,
            pltpu.SemaphoreType.DMA,
        ],
        name="sc_gcn_degree",
    )
    def deg_k(dst_hbm, out_hbm, dba, dbb, vals, acc, semia, semib, sems0, sems1):
        cid = lax.axis_index("c")
        sid = lax.axis_index("s")
        rowbase = (cid * NS + sid) * nch
        for k in range(ch // 16):
            vals[pl.ds(k * 16, 16)] = jnp.zeros((16,), jnp.float32)
        for r in range(rpw // ch):
            pltpu.async_copy(vals, acc.at[pl.ds(sid * rpw + r * ch, ch)], sems0)
        for r in range(rpw // ch):
            pltpu.make_async_copy(vals, acc.at[pl.ds(sid * rpw, ch)], sems0).wait()
        plsc.subcore_barrier()
        for k in range(ch // 16):
            vals[pl.ds(k * 16, 16)] = jnp.ones((16,), jnp.float32)

        pltpu.async_copy(dst_hbm.at[pl.ds(rowbase, bs)], dba, semia)
        pltpu.async_copy(dst_hbm.at[pl.ds(rowbase + bs, bs)], dbb, semib)

        def run_block(db, semi, bnext):
            pltpu.make_async_copy(dst_hbm.at[pl.ds(rowbase, bs)], db, semi).wait()

            def inner(i, carry):
                c0 = 2 * i

                @pl.when(i > 0)
                def _():
                    pltpu.make_async_copy(vals, acc.at[db.at[0]], sems0).wait()
                    pltpu.make_async_copy(vals, acc.at[db.at[0]], sems1).wait()

                pltpu.async_copy(vals, acc.at[db.at[c0]], sems0, add=True)
                pltpu.async_copy(vals, acc.at[db.at[c0 + 1]], sems1, add=True)
                return carry

            lax.fori_loop(0, bs // 2, inner, 0)
            pltpu.make_async_copy(vals, acc.at[db.at[0]], sems0).wait()
            pltpu.make_async_copy(vals, acc.at[db.at[0]], sems1).wait()
            pltpu.async_copy(dst_hbm.at[pl.ds(rowbase + bnext * bs, bs)], db, semi)

        def outer(bb, carry):
            b0 = 2 * bb
            run_block(dba, semia, jnp.minimum(b0 + 2, nblk - 1))
            run_block(dbb, semib, jnp.minimum(b0 + 3, nblk - 1))
            return carry

        lax.fori_loop(0, nblk // 2, outer, 0)
        pltpu.make_async_copy(dst_hbm.at[pl.ds(rowbase, bs)], dba, semia).wait()
        pltpu.make_async_copy(dst_hbm.at[pl.ds(rowbase, bs)], dbb, semib).wait()
        plsc.subcore_barrier()
        pltpu.sync_copy(acc.at[pl.ds(sid * rpw, rpw)],
                        out_hbm.at[pl.ds(cid * rows + sid * rpw, rpw)])

    return deg_k(dst2d)


def _sc_prop(table, src2d, dst2d, rows, rpw, nblk, bs, ch, d):
    nch = nblk * bs
    gshape = (ch,) if d is None else (ch, d)
    oshape = (NC * rows,) if d is None else (NC * rows, d)
    dd = 1 if d is None else d

    @functools.partial(
        pl.kernel,
        out_type=jax.ShapeDtypeStruct(oshape, jnp.float32),
        mesh=_mesh(),
        scratch_types=[
            pltpu.VMEM((bs, ch), jnp.int32),
            pltpu.VMEM((bs, ch), jnp.int32),
            pltpu.VMEM((bs, ch), jnp.int32),
            pltpu.VMEM((bs, ch), jnp.int32),
            pltpu.VMEM(gshape, jnp.float32),
            pltpu.VMEM(gshape, jnp.float32),
            pltpu.VMEM_SHARED((rows,) if d is None else (rows, d), jnp.float32),
            pltpu.SemaphoreType.DMA,
            pltpu.SemaphoreType.DMA,
            pltpu.SemaphoreType.DMA,
            pltpu.SemaphoreType.DMA,
            pltpu.SemaphoreType.DMA,
            pltpu.SemaphoreType.DMA,
        ],
        name="sc_gcn_prop%d" % dd,
    )
    def prop_k(tab_hbm, src_hbm, dst_hbm, out_hbm,
               sba, sbb, dba, dbb, g0, g1, acc,
               semia, semib, semg0, semg1, sems0, sems1):
        cid = lax.axis_index("c")
        sid = lax.axis_index("s")
        rowbase = (cid * NS + sid) * nch

        if d is None:
            for k in range(ch // 16):
                g0[pl.ds(k * 16, 16)] = jnp.zeros((16,), jnp.float32)
        else:
            def zbody(r, carry):
                for k in range(d // 16):
                    g0[r, pl.ds(k * 16, 16)] = jnp.zeros((16,), jnp.float32)
                return carry

            lax.fori_loop(0, ch, zbody, 0)
        for r in range(rpw // ch):
            pltpu.async_copy(g0, acc.at[pl.ds(sid * rpw + r * ch, ch)], sems0)
        for r in range(rpw // ch):
            pltpu.make_async_copy(g0, acc.at[pl.ds(sid * rpw, ch)], sems0).wait()
        plsc.subcore_barrier()

        pltpu.async_copy(src_hbm.at[pl.ds(rowbase, bs)], sba, semia)
        pltpu.async_copy(dst_hbm.at[pl.ds(rowbase, bs)], dba, semia)
        pltpu.async_copy(src_hbm.at[pl.ds(rowbase + bs, bs)], sbb, semib)
        pltpu.async_copy(dst_hbm.at[pl.ds(rowbase + bs, bs)], dbb, semib)

        def run_block(sb, db, semi, bnext):
            pltpu.make_async_copy(src_hbm.at[pl.ds(rowbase, bs)], sb, semi).wait()
            pltpu.make_async_copy(dst_hbm.at[pl.ds(rowbase, bs)], db, semi).wait()

            def inner(i, carry):
                c0 = 2 * i

                @pl.when(i > 0)
                def _():
                    pltpu.make_async_copy(g0, acc.at[db.at[0]], sems0).wait()
                    pltpu.make_async_copy(g1, acc.at[db.at[0]], sems1).wait()

                pltpu.async_copy(tab_hbm.at[sb.at[c0]], g0, semg0)
                pltpu.async_copy(tab_hbm.at[sb.at[c0 + 1]], g1, semg1)
                pltpu.make_async_copy(tab_hbm.at[sb.at[c0]], g0, semg0).wait()
                pltpu.async_copy(g0, acc.at[db.at[c0]], sems0, add=True)
                pltpu.make_async_copy(tab_hbm.at[sb.at[c0]], g1, semg1).wait()
                pltpu.async_copy(g1, acc.at[db.at[c0 + 1]], sems1, add=True)
                return carry

            lax.fori_loop(0, bs // 2, inner, 0)
            pltpu.make_async_copy(g0, acc.at[db.at[0]], sems0).wait()
            pltpu.make_async_copy(g1, acc.at[db.at[0]], sems1).wait()
            pltpu.async_copy(src_hbm.at[pl.ds(rowbase + bnext * bs, bs)], sb, semi)
            pltpu.async_copy(dst_hbm.at[pl.ds(rowbase + bnext * bs, bs)], db, semi)

        def outer(bb, carry):
            b0 = 2 * bb
            run_block(sba, dba, semia, jnp.minimum(b0 + 2, nblk - 1))
            run_block(sbb, dbb, semib, jnp.minimum(b0 + 3, nblk - 1))
            return carry

        lax.fori_loop(0, nblk // 2, outer, 0)
        pltpu.make_async_copy(src_hbm.at[pl.ds(rowbase, bs)], sba, semia).wait()
        pltpu.make_async_copy(dst_hbm.at[pl.ds(rowbase, bs)], dba, semia).wait()
        pltpu.make_async_copy(src_hbm.at[pl.ds(rowbase, bs)], sbb, semib).wait()
        pltpu.make_async_copy(dst_hbm.at[pl.ds(rowbase, bs)], dbb, semib).wait()
        plsc.subcore_barrier()
        pltpu.sync_copy(acc.at[pl.ds(sid * rpw, rpw)],
                        out_hbm.at[pl.ds(cid * rows + sid * rpw, rpw)])

    return prop_k(table, src2d, dst2d)


def _tc_scale_matmul(x, w1, d0, d1, n, d_hid):
    def body(x_ref, w_ref, d0_ref, d1_ref, h_ref, dinv_ref):
        deg = d0_ref[...] + d1_ref[...] + 1.0
        dinv = lax.rsqrt(jnp.maximum(deg, 1.0))
        h = jnp.dot(x_ref[...], w_ref[...], preferred_element_type=jnp.float32)
        h_ref[...] = h * dinv
        dinv_ref[...] = dinv

    return pl.pallas_call(
        body,
        out_shape=(jax.ShapeDtypeStruct((n, d_hid), jnp.float32),
                   jax.ShapeDtypeStruct((n, 1), jnp.float32)),
    )(x, w1, d0, d1)


def _tc_layer2_in(p0, p1, h1s, dinv, b1, w2, n):
    def body(p0_ref, p1_ref, h_ref, dinv_ref, b1_ref, w2_ref, vs_ref):
        out1 = dinv_ref[...] * (p0_ref[...] + p1_ref[...] + h_ref[...]) + b1_ref[...]
        hrelu = jnp.maximum(out1, 0.0)
        v = jnp.dot(hrelu, w2_ref[...], preferred_element_type=jnp.float32)
        vs_ref[...] = dinv_ref[...] * v

    return pl.pallas_call(
        body,
        out_shape=jax.ShapeDtypeStruct((n, 1), jnp.float32),
    )(p0, p1, h1s, dinv, b1, w2)


def _tc_finish(t0, t1, vs, dinv, b2, n):
    def body(t0_ref, t1_ref, vs_ref, dinv_ref, b2_ref, o_ref):
        z = dinv_ref[...] * (t0_ref[...] + t1_ref[...] + vs_ref[...]) + b2_ref[...]
        o_ref[...] = 1.0 / (1.0 + jnp.exp(-z))

    return pl.pallas_call(
        body,
        out_shape=jax.ShapeDtypeStruct((n, 1), jnp.float32),
    )(t0, t1, vs, dinv, b2)


def kernel(x, edge_index, W1, b1, W2, b2):
    n, d_in = x.shape
    d_hid = W1.shape[1]
    e = edge_index.shape[1]

    nw = NC * NS
    chw, bsw = 64, 16
    nblkw = (-(-e // (nw * chw)) + bsw - 1) // bsw
    nblkw += nblkw % 2
    epw = nblkw * bsw * chw
    chs, bss = 128, 8
    nblks = (epw // chs + bss - 1) // bss
    assert nblks % 2 == 0 and nblks * bss * chs == epw, (nblks, epw)
    e_pad = epw * nw
    rpw = -(-(n + 1) // (NS * chs)) * chs
    rows = rpw * NS

    src = edge_index[0]
    dst = edge_index[1]
    pad = e_pad - e
    src_pad = jnp.concatenate([src, jnp.zeros((pad,), jnp.int32)])
    dst_pad = jnp.concatenate([dst, jnp.full((pad,), n, jnp.int32)])
    srcw = src_pad.reshape(nw * nblkw * bsw, chw)
    dstw = dst_pad.reshape(nw * nblkw * bsw, chw)
    srcs = src_pad.reshape(nw * nblks * bss, chs)
    dsts = dst_pad.reshape(nw * nblks * bss, chs)

    deg_parts = _sc_degree(dsts, rows, rpw, nblks, bss, chs)
    d0 = deg_parts[:n].reshape(n, 1)
    d1 = deg_parts[rows:rows + n].reshape(n, 1)

    h1s, dinv = _tc_scale_matmul(x, W1, d0, d1, n, d_hid)

    parts = _sc_prop(h1s, srcw, dstw, rows, rpw, nblkw, bsw, chw, d_hid)
    p0 = parts[:n]
    p1 = parts[rows:rows + n]

    vs = _tc_layer2_in(p0, p1, h1s, dinv, b1.reshape(1, d_hid), W2, n)

    t_parts = _sc_prop(vs.reshape(n), srcs, dsts, rows, rpw, nblks, bss, chs, None)
    t0 = t_parts[:n].reshape(n, 1)
    t1 = t_parts[rows:rows + n].reshape(n, 1)

    out = _tc_finish(t0, t1, vs, dinv, b2.reshape(1, 1), n)
    return out.reshape(n)


# speedup vs baseline: 1.2704x; 1.2704x over previous
import functools

import jax
import jax.numpy as jnp
from jax import lax
from jax.experimental import pallas as pl
from jax.experimental.pallas import tpu as pltpu
from jax.experimental.pallas import tpu_sc as plsc

NC = 2
NS = 16


def _mesh():
    return plsc.VectorSubcoreMesh(core_axis_name="c", subcore_axis_name="s")


def _sc_degree(dst2d, rows, rpw, nblk, bs, ch):
    nch = nblk * bs

    @functools.partial(
        pl.kernel,
        out_type=jax.ShapeDtypeStruct((NC * rows,), jnp.float32),
        mesh=_mesh(),
        scratch_types=[
            pltpu.VMEM((bs, ch), jnp.int32),
            pltpu.VMEM((bs, ch), jnp.int32),
            pltpu.VMEM((ch,), jnp.float32),
            pltpu.VMEM_SHARED((rows,), jnp.float32),
            pltpu.SemaphoreType.DMA,
            pltpu.SemaphoreType.DMA,
            pltpu.SemaphoreType.DMA,
            pltpu.SemaphoreType.DMA,
        ],
        name="sc_gcn_degree",
    )
    def deg_k(dst_hbm, out_hbm, dba, dbb, vals, acc, semia, semib, sems0, sems1):
        cid = lax.axis_index("c")
        sid = lax.axis_index("s")
        rowbase = (cid * NS + sid) * nch
        for k in range(ch // 16):
            vals[pl.ds(k * 16, 16)] = jnp.zeros((16,), jnp.float32)
        for r in range(rpw // ch):
            pltpu.async_copy(vals, acc.at[pl.ds(sid * rpw + r * ch, ch)], sems0)
        for r in range(rpw // ch):
            pltpu.make_async_copy(vals, acc.at[pl.ds(sid * rpw, ch)], sems0).wait()
        plsc.subcore_barrier()
        for k in range(ch // 16):
            vals[pl.ds(k * 16, 16)] = jnp.ones((16,), jnp.float32)

        pltpu.async_copy(dst_hbm.at[pl.ds(rowbase, bs)], dba, semia)
        pltpu.async_copy(dst_hbm.at[pl.ds(rowbase + bs, bs)], dbb, semib)

        def run_block(db, semi, bnext):
            pltpu.make_async_copy(dst_hbm.at[pl.ds(rowbase, bs)], db, semi).wait()

            def inner(i, carry):
                c0 = 2 * i

                @pl.when(i > 0)
                def _():
                    pltpu.make_async_copy(vals, acc.at[db.at[0]], sems0).wait()
                    pltpu.make_async_copy(vals, acc.at[db.at[0]], sems1).wait()

                pltpu.async_copy(vals, acc.at[db.at[c0]], sems0, add=True)
                pltpu.async_copy(vals, acc.at[db.at[c0 + 1]], sems1, add=True)
                return carry

            lax.fori_loop(0, bs // 2, inner, 0)
            pltpu.make_async_copy(vals, acc.at[db.at[0]], sems0).wait()
            pltpu.make_async_copy(vals, acc.at[db.at[0]], sems1).wait()
            pltpu.async_copy(dst_hbm.at[pl.ds(rowbase + bnext * bs, bs)], db, semi)

        def outer(bb, carry):
            b0 = 2 * bb
            run_block(dba, semia, jnp.minimum(b0 + 2, nblk - 1))
            run_block(dbb, semib, jnp.minimum(b0 + 3, nblk - 1))
            return carry

        lax.fori_loop(0, nblk // 2, outer, 0)
        pltpu.make_async_copy(dst_hbm.at[pl.ds(rowbase, bs)], dba, semia).wait()
        pltpu.make_async_copy(dst_hbm.at[pl.ds(rowbase, bs)], dbb, semib).wait()
        plsc.subcore_barrier()
        pltpu.sync_copy(acc.at[pl.ds(sid * rpw, rpw)],
                        out_hbm.at[pl.ds(cid * rows + sid * rpw, rpw)])

    return deg_k(dst2d)


def _sc_prop_wide(table, src_pad, dst_pad, rows, rpw, nch, epw, ch, d):
    @functools.partial(
        pl.kernel,
        out_type=jax.ShapeDtypeStruct((NC * rows, d), jnp.float32),
        mesh=_mesh(),
        scratch_types=[
            pltpu.VMEM((ch,), jnp.int32),
            pltpu.VMEM((ch,), jnp.int32),
            pltpu.VMEM((ch,), jnp.int32),
            pltpu.VMEM((ch,), jnp.int32),
            pltpu.VMEM((ch, d), jnp.float32),
            pltpu.VMEM((ch, d), jnp.float32),
            pltpu.VMEM_SHARED((rows, d), jnp.float32),
            pltpu.SemaphoreType.DMA,
            pltpu.SemaphoreType.DMA,
            pltpu.SemaphoreType.DMA,
            pltpu.SemaphoreType.DMA,
        ],
        name="sc_gcn_prop128",
    )
    def prop_k(tab_hbm, src_hbm, dst_hbm, out_hbm,
               sb0, sb1, db0, db1, g0, g1, acc,
               semi0, semi1, semg0, semg1):
        cid = lax.axis_index("c")
        sid = lax.axis_index("s")
        base = (cid * NS + sid) * epw

        def zbody(r, carry):
            for k in range(d // 16):
                g0[r, pl.ds(k * 16, 16)] = jnp.zeros((16,), jnp.float32)
            return carry

        lax.fori_loop(0, ch, zbody, 0)
        for r in range(rpw // ch):
            pltpu.sync_copy(g0, acc.at[pl.ds(sid * rpw + r * ch, ch)])
        plsc.subcore_barrier()

        pltpu.async_copy(src_hbm.at[pl.ds(base, ch)], sb0, semi0)
        pltpu.async_copy(dst_hbm.at[pl.ds(base, ch)], db0, semi0)
        pltpu.async_copy(src_hbm.at[pl.ds(base + ch, ch)], sb1, semi1)
        pltpu.async_copy(dst_hbm.at[pl.ds(base + ch, ch)], db1, semi1)
        pltpu.make_async_copy(src_hbm.at[pl.ds(base, ch)], sb0, semi0).wait()
        pltpu.make_async_copy(dst_hbm.at[pl.ds(base, ch)], db0, semi0).wait()
        pltpu.async_copy(tab_hbm.at[sb0], g0, semg0)

        def body(jj, carry):
            j = jj * 2
            o2 = base + jnp.minimum(j + 2, nch - 1) * ch
            o3 = base + jnp.minimum(j + 3, nch - 1) * ch
            pltpu.make_async_copy(src_hbm.at[pl.ds(base, ch)], sb1, semi1).wait()
            pltpu.make_async_copy(dst_hbm.at[pl.ds(base, ch)], db1, semi1).wait()
            pltpu.async_copy(tab_hbm.at[sb1], g1, semg1)
            pltpu.make_async_copy(tab_hbm.at[sb0], g0, semg0).wait()
            pltpu.sync_copy(g0, acc.at[db0], add=True)
            pltpu.async_copy(src_hbm.at[pl.ds(o2, ch)], sb0, semi0)
            pltpu.async_copy(dst_hbm.at[pl.ds(o2, ch)], db0, semi0)
            pltpu.make_async_copy(src_hbm.at[pl.ds(base, ch)], sb0, semi0).wait()
            pltpu.make_async_copy(dst_hbm.at[pl.ds(base, ch)], db0, semi0).wait()
            pltpu.async_copy(tab_hbm.at[sb0], g0, semg0)
            pltpu.make_async_copy(tab_hbm.at[sb1], g1, semg1).wait()
            pltpu.sync_copy(g1, acc.at[db1], add=True)
            pltpu.async_copy(src_hbm.at[pl.ds(o3, ch)], sb1, semi1)
            pltpu.async_copy(dst_hbm.at[pl.ds(o3, ch)], db1, semi1)
            return carry

        lax.fori_loop(0, nch // 2, body, 0)
        pltpu.make_async_copy(tab_hbm.at[sb0], g0, semg0).wait()
        pltpu.make_async_copy(src_hbm.at[pl.ds(base, ch)], sb1, semi1).wait()
        pltpu.make_async_copy(dst_hbm.at[pl.ds(base, ch)], db1, semi1).wait()
        plsc.subcore_barrier()
        pltpu.sync_copy(acc.at[pl.ds(sid * rpw, rpw)],
                        out_hbm.at[pl.ds(cid * rows + sid * rpw, rpw)])

    return prop_k(table, src_pad, dst_pad)


def _sc_prop_scalar(vec, src2d, dst2d, rows, rpw, nblk, bs, ch):
    nch = nblk * bs

    @functools.partial(
        pl.kernel,
        out_type=jax.ShapeDtypeStruct((NC * rows,), jnp.float32),
        mesh=_mesh(),
        scratch_types=[
            pltpu.VMEM((bs, ch), jnp.int32),
            pltpu.VMEM((bs, ch), jnp.int32),
            pltpu.VMEM((bs, ch), jnp.int32),
            pltpu.VMEM((bs, ch), jnp.int32),
            pltpu.VMEM((ch,), jnp.float32),
            pltpu.VMEM((ch,), jnp.float32),
            pltpu.VMEM_SHARED((rows,), jnp.float32),
            pltpu.SemaphoreType.DMA,
            pltpu.SemaphoreType.DMA,
            pltpu.SemaphoreType.DMA,
            pltpu.SemaphoreType.DMA,
            pltpu.SemaphoreType.DMA,
            pltpu.SemaphoreType.DMA,
        ],
        name="sc_gcn_prop1",
    )
    def prop1_k(vec_hbm, src_hbm, dst_hbm, out_hbm,
                sba, sbb, dba, dbb, g0, g1, acc,
                semia, semib, semg0, semg1, sems0, sems1):
        cid = lax.axis_index("c")
        sid = lax.axis_index("s")
        rowbase = (cid * NS + sid) * nch
        for k in range(ch // 16):
            g0[pl.ds(k * 16, 16)] = jnp.zeros((16,), jnp.float32)
        for r in range(rpw // ch):
            pltpu.async_copy(g0, acc.at[pl.ds(sid * rpw + r * ch, ch)], sems0)
        for r in range(rpw // ch):
            pltpu.make_async_copy(g0, acc.at[pl.ds(sid * rpw, ch)], sems0).wait()
        plsc.subcore_barrier()

        pltpu.async_copy(src_hbm.at[pl.ds(rowbase, bs)], sba, semia)
        pltpu.async_copy(dst_hbm.at[pl.ds(rowbase, bs)], dba, semia)
        pltpu.async_copy(src_hbm.at[pl.ds(rowbase + bs, bs)], sbb, semib)
        pltpu.async_copy(dst_hbm.at[pl.ds(rowbase + bs, bs)], dbb, semib)

        def run_block(sb, db, semi, bnext):
            pltpu.make_async_copy(src_hbm.at[pl.ds(rowbase, bs)], sb, semi).wait()
            pltpu.make_async_copy(dst_hbm.at[pl.ds(rowbase, bs)], db, semi).wait()

            def inner(i, carry):
                c0 = 2 * i

                @pl.when(i > 0)
                def _():
                    pltpu.make_async_copy(g0, acc.at[db.at[0]], sems0).wait()
                    pltpu.make_async_copy(g1, acc.at[db.at[0]], sems1).wait()

                pltpu.async_copy(vec_hbm.at[sb.at[c0]], g0, semg0)
                pltpu.async_copy(vec_hbm.at[sb.at[c0 + 1]], g1, semg1)
                pltpu.make_async_copy(vec_hbm.at[sb.at[c0]], g0, semg0).wait()
                pltpu.async_copy(g0, acc.at[db.at[c0]], sems0, add=True)
                pltpu.make_async_copy(vec_hbm.at[sb.at[c0]], g1, semg1).wait()
                pltpu.async_copy(g1, acc.at[db.at[c0 + 1]], sems1, add=True)
                return carry

            lax.fori_loop(0, bs // 2, inner, 0)
            pltpu.make_async_copy(g0, acc.at[db.at[0]], sems0).wait()
            pltpu.make_async_copy(g1, acc.at[db.at[0]], sems1).wait()
            pltpu.async_copy(src_hbm.at[pl.ds(rowbase + bnext * bs, bs)], sb, semi)
            pltpu.async_copy(dst_hbm.at[pl.ds(rowbase + bnext * bs, bs)], db, semi)

        def outer(bb, carry):
            b0 = 2 * bb
            run_block(sba, dba, semia, jnp.minimum(b0 + 2, nblk - 1))
            run_block(sbb, dbb, semib, jnp.minimum(b0 + 3, nblk - 1))
            return carry

        lax.fori_loop(0, nblk // 2, outer, 0)
        pltpu.make_async_copy(src_hbm.at[pl.ds(rowbase, bs)], sba, semia).wait()
        pltpu.make_async_copy(dst_hbm.at[pl.ds(rowbase, bs)], dba, semia).wait()
        pltpu.make_async_copy(src_hbm.at[pl.ds(rowbase, bs)], sbb, semib).wait()
        pltpu.make_async_copy(dst_hbm.at[pl.ds(rowbase, bs)], dbb, semib).wait()
        plsc.subcore_barrier()
        pltpu.sync_copy(acc.at[pl.ds(sid * rpw, rpw)],
                        out_hbm.at[pl.ds(cid * rows + sid * rpw, rpw)])

    return prop1_k(vec, src2d, dst2d)


def _tc_scale_matmul(x, w1, d0, d1, n, d_hid):
    def body(x_ref, w_ref, d0_ref, d1_ref, h_ref, dinv_ref):
        deg = d0_ref[...] + d1_ref[...] + 1.0
        dinv = lax.rsqrt(jnp.maximum(deg, 1.0))
        h = jnp.dot(x_ref[...], w_ref[...], preferred_element_type=jnp.float32)
        h_ref[...] = h * dinv
        dinv_ref[...] = dinv

    return pl.pallas_call(
        body,
        out_shape=(jax.ShapeDtypeStruct((n, d_hid), jnp.float32),
                   jax.ShapeDtypeStruct((n, 1), jnp.float32)),
    )(x, w1, d0, d1)


def _tc_layer2_in(p0, p1, h1s, dinv, b1, w2, n):
    def body(p0_ref, p1_ref, h_ref, dinv_ref, b1_ref, w2_ref, vs_ref):
        out1 = dinv_ref[...] * (p0_ref[...] + p1_ref[...] + h_ref[...]) + b1_ref[...]
        hrelu = jnp.maximum(out1, 0.0)
        v = jnp.dot(hrelu, w2_ref[...], preferred_element_type=jnp.float32)
        vs_ref[...] = dinv_ref[...] * v

    return pl.pallas_call(
        body,
        out_shape=jax.ShapeDtypeStruct((n, 1), jnp.float32),
    )(p0, p1, h1s, dinv, b1, w2)


def _tc_finish(t0, t1, vs, dinv, b2, n):
    def body(t0_ref, t1_ref, vs_ref, dinv_ref, b2_ref, o_ref):
        z = dinv_ref[...] * (t0_ref[...] + t1_ref[...] + vs_ref[...]) + b2_ref[...]
        o_ref[...] = 1.0 / (1.0 + jnp.exp(-z))

    return pl.pallas_call(
        body,
        out_shape=jax.ShapeDtypeStruct((n, 1), jnp.float32),
    )(t0, t1, vs, dinv, b2)


def kernel(x, edge_index, W1, b1, W2, b2):
    n, d_in = x.shape
    d_hid = W1.shape[1]
    e = edge_index.shape[1]

    nw = NC * NS
    chw = 64
    nchw = -(-e // (nw * chw))
    nchw = -(-nchw // 8) * 8
    epw = nchw * chw
    e_pad = epw * nw
    chs, bss = 128, 8
    nblks = epw // chs // bss
    assert nblks % 2 == 0 and nblks * bss * chs == epw, (nblks, epw)
    rpw = -(-(n + 1) // (NS * chs)) * chs
    rows = rpw * NS

    src = edge_index[0]
    dst = edge_index[1]
    pad = e_pad - e
    src_pad = jnp.concatenate([src, jnp.zeros((pad,), jnp.int32)])
    dst_pad = jnp.concatenate([dst, jnp.full((pad,), n, jnp.int32)])
    srcs = src_pad.reshape(nw * nblks * bss, chs)
    dsts = dst_pad.reshape(nw * nblks * bss, chs)

    deg_parts = _sc_degree(dsts, rows, rpw, nblks, bss, chs)
    d0 = deg_parts[:n].reshape(n, 1)
    d1 = deg_parts[rows:rows + n].reshape(n, 1)

    h1s, dinv = _tc_scale_matmul(x, W1, d0, d1, n, d_hid)

    parts = _sc_prop_wide(h1s, src_pad, dst_pad, rows, rpw, nchw, epw, chw, d_hid)
    p0 = parts[:n]
    p1 = parts[rows:rows + n]

    vs = _tc_layer2_in(p0, p1, h1s, dinv, b1.reshape(1, d_hid), W2, n)

    t_parts = _sc_prop_scalar(vs.reshape(n), srcs, dsts, rows, rpw, nblks, bss, chs)
    t0 = t_parts[:n].reshape(n, 1)
    t1 = t_parts[rows:rows + n].reshape(n, 1)

    out = _tc_finish(t0, t1, vs, dinv, b2.reshape(1, 1), n)
    return out.reshape(n)


# speedup vs baseline: 2.5129x; 1.9780x over previous
import functools

import jax
import jax.numpy as jnp
from jax import lax
from jax.experimental import pallas as pl
from jax.experimental.pallas import tpu as pltpu
from jax.experimental.pallas import tpu_sc as plsc

NC = 2
NS = 16


def _mesh():
    return plsc.VectorSubcoreMesh(core_axis_name="c", subcore_axis_name="s")


def _sc_degree(dst2d, rows, rpw, nblk, bs, ch):
    nch = nblk * bs

    @functools.partial(
        pl.kernel,
        out_type=jax.ShapeDtypeStruct((NC * rows,), jnp.float32),
        mesh=_mesh(),
        scratch_types=[
            pltpu.VMEM((bs, ch), jnp.int32),
            pltpu.VMEM((bs, ch), jnp.int32),
            pltpu.VMEM((ch,), jnp.float32),
            pltpu.VMEM_SHARED((rows,), jnp.float32),
            pltpu.SemaphoreType.DMA,
            pltpu.SemaphoreType.DMA,
            pltpu.SemaphoreType.DMA,
            pltpu.SemaphoreType.DMA,
        ],
        name="sc_gcn_degree",
    )
    def deg_k(dst_hbm, out_hbm, dba, dbb, vals, acc, semia, semib, sems0, sems1):
        cid = lax.axis_index("c")
        sid = lax.axis_index("s")
        rowbase = (cid * NS + sid) * nch
        for k in range(ch // 16):
            vals[pl.ds(k * 16, 16)] = jnp.zeros((16,), jnp.float32)
        for r in range(rpw // ch):
            pltpu.async_copy(vals, acc.at[pl.ds(sid * rpw + r * ch, ch)], sems0)
        for r in range(rpw // ch):
            pltpu.make_async_copy(vals, acc.at[pl.ds(sid * rpw, ch)], sems0).wait()
        plsc.subcore_barrier()
        for k in range(ch // 16):
            vals[pl.ds(k * 16, 16)] = jnp.ones((16,), jnp.float32)

        pltpu.async_copy(dst_hbm.at[pl.ds(rowbase, bs)], dba, semia)
        pltpu.async_copy(dst_hbm.at[pl.ds(rowbase + bs, bs)], dbb, semib)

        def run_block(db, semi, bnext):
            pltpu.make_async_copy(dst_hbm.at[pl.ds(rowbase, bs)], db, semi).wait()

            def inner(i, carry):
                c0 = 2 * i

                @pl.when(i > 0)
                def _():
                    pltpu.make_async_copy(vals, acc.at[db.at[0]], sems0).wait()
                    pltpu.make_async_copy(vals, acc.at[db.at[0]], sems1).wait()

                pltpu.async_copy(vals, acc.at[db.at[c0]], sems0, add=True)
                pltpu.async_copy(vals, acc.at[db.at[c0 + 1]], sems1, add=True)
                return carry

            lax.fori_loop(0, bs // 2, inner, 0)
            pltpu.make_async_copy(vals, acc.at[db.at[0]], sems0).wait()
            pltpu.make_async_copy(vals, acc.at[db.at[0]], sems1).wait()
            pltpu.async_copy(dst_hbm.at[pl.ds(rowbase + bnext * bs, bs)], db, semi)

        def outer(bb, carry):
            b0 = 2 * bb
            run_block(dba, semia, jnp.minimum(b0 + 2, nblk - 1))
            run_block(dbb, semib, jnp.minimum(b0 + 3, nblk - 1))
            return carry

        lax.fori_loop(0, nblk // 2, outer, 0)
        pltpu.make_async_copy(dst_hbm.at[pl.ds(rowbase, bs)], dba, semia).wait()
        pltpu.make_async_copy(dst_hbm.at[pl.ds(rowbase, bs)], dbb, semib).wait()
        plsc.subcore_barrier()
        pltpu.sync_copy(acc.at[pl.ds(sid * rpw, rpw)],
                        out_hbm.at[pl.ds(cid * rows + sid * rpw, rpw)])

    return deg_k(dst2d)


def _sc_prop_wide(table, src_pad, dst_pad, rows, rpw, nch, epw, ch, d):
    @functools.partial(
        pl.kernel,
        out_type=jax.ShapeDtypeStruct((NC * rows, d), jnp.float32),
        mesh=_mesh(),
        scratch_types=[
            pltpu.VMEM((ch,), jnp.int32),
            pltpu.VMEM((ch,), jnp.int32),
            pltpu.VMEM((ch,), jnp.int32),
            pltpu.VMEM((ch,), jnp.int32),
            pltpu.VMEM((ch, d), jnp.float32),
            pltpu.VMEM((ch, d), jnp.float32),
            pltpu.VMEM_SHARED((rows, d), jnp.float32),
            pltpu.SemaphoreType.DMA,
            pltpu.SemaphoreType.DMA,
            pltpu.SemaphoreType.DMA,
            pltpu.SemaphoreType.DMA,
        ],
        name="sc_gcn_prop128",
    )
    def prop_k(tab_hbm, src_hbm, dst_hbm, out_hbm,
               sb0, sb1, db0, db1, g0, g1, acc,
               semi0, semi1, semg0, semg1):
        cid = lax.axis_index("c")
        sid = lax.axis_index("s")
        base = (cid * NS + sid) * epw

        def zbody(r, carry):
            for k in range(d // 16):
                g0[r, pl.ds(k * 16, 16)] = jnp.zeros((16,), jnp.float32)
            return carry

        lax.fori_loop(0, ch, zbody, 0)
        for r in range(rpw // ch):
            pltpu.sync_copy(g0, acc.at[pl.ds(sid * rpw + r * ch, ch)])
        plsc.subcore_barrier()

        pltpu.async_copy(src_hbm.at[pl.ds(base, ch)], sb0, semi0)
        pltpu.async_copy(dst_hbm.at[pl.ds(base, ch)], db0, semi0)
        pltpu.async_copy(src_hbm.at[pl.ds(base + ch, ch)], sb1, semi1)
        pltpu.async_copy(dst_hbm.at[pl.ds(base + ch, ch)], db1, semi1)
        pltpu.make_async_copy(src_hbm.at[pl.ds(base, ch)], sb0, semi0).wait()
        pltpu.make_async_copy(dst_hbm.at[pl.ds(base, ch)], db0, semi0).wait()
        pltpu.async_copy(tab_hbm.at[sb0], g0, semg0)

        def body(jj, carry):
            j = jj * 2
            o2 = base + jnp.minimum(j + 2, nch - 1) * ch
            o3 = base + jnp.minimum(j + 3, nch - 1) * ch
            pltpu.make_async_copy(src_hbm.at[pl.ds(base, ch)], sb1, semi1).wait()
            pltpu.make_async_copy(dst_hbm.at[pl.ds(base, ch)], db1, semi1).wait()
            pltpu.async_copy(tab_hbm.at[sb1], g1, semg1)
            pltpu.make_async_copy(tab_hbm.at[sb0], g0, semg0).wait()
            pltpu.sync_copy(g0, acc.at[db0], add=True)
            pltpu.async_copy(src_hbm.at[pl.ds(o2, ch)], sb0, semi0)
            pltpu.async_copy(dst_hbm.at[pl.ds(o2, ch)], db0, semi0)
            pltpu.make_async_copy(src_hbm.at[pl.ds(base, ch)], sb0, semi0).wait()
            pltpu.make_async_copy(dst_hbm.at[pl.ds(base, ch)], db0, semi0).wait()
            pltpu.async_copy(tab_hbm.at[sb0], g0, semg0)
            pltpu.make_async_copy(tab_hbm.at[sb1], g1, semg1).wait()
            pltpu.sync_copy(g1, acc.at[db1], add=True)
            pltpu.async_copy(src_hbm.at[pl.ds(o3, ch)], sb1, semi1)
            pltpu.async_copy(dst_hbm.at[pl.ds(o3, ch)], db1, semi1)
            return carry

        lax.fori_loop(0, nch // 2, body, 0)
        pltpu.make_async_copy(tab_hbm.at[sb0], g0, semg0).wait()
        pltpu.make_async_copy(src_hbm.at[pl.ds(base, ch)], sb1, semi1).wait()
        pltpu.make_async_copy(dst_hbm.at[pl.ds(base, ch)], db1, semi1).wait()
        plsc.subcore_barrier()
        pltpu.sync_copy(acc.at[pl.ds(sid * rpw, rpw)],
                        out_hbm.at[pl.ds(cid * rows + sid * rpw, rpw)])

    return prop_k(table, src_pad, dst_pad)


def _sc_prop_scalar(vec, src2d, dst2d, rows, rpw, nblk, bs, ch):
    nch = nblk * bs

    @functools.partial(
        pl.kernel,
        out_type=jax.ShapeDtypeStruct((NC * rows,), jnp.float32),
        mesh=_mesh(),
        scratch_types=[
            pltpu.VMEM((bs, ch), jnp.int32),
            pltpu.VMEM((bs, ch), jnp.int32),
            pltpu.VMEM((bs, ch), jnp.int32),
            pltpu.VMEM((bs, ch), jnp.int32),
            pltpu.VMEM((ch,), jnp.float32),
            pltpu.VMEM((ch,), jnp.float32),
            pltpu.VMEM_SHARED((rows,), jnp.float32),
            pltpu.SemaphoreType.DMA,
            pltpu.SemaphoreType.DMA,
            pltpu.SemaphoreType.DMA,
            pltpu.SemaphoreType.DMA,
            pltpu.SemaphoreType.DMA,
            pltpu.SemaphoreType.DMA,
        ],
        name="sc_gcn_prop1",
    )
    def prop1_k(vec_hbm, src_hbm, dst_hbm, out_hbm,
                sba, sbb, dba, dbb, g0, g1, acc,
                semia, semib, semg0, semg1, sems0, sems1):
        cid = lax.axis_index("c")
        sid = lax.axis_index("s")
        rowbase = (cid * NS + sid) * nch
        for k in range(ch // 16):
            g0[pl.ds(k * 16, 16)] = jnp.zeros((16,), jnp.float32)
        for r in range(rpw // ch):
            pltpu.async_copy(g0, acc.at[pl.ds(sid * rpw + r * ch, ch)], sems0)
        for r in range(rpw // ch):
            pltpu.make_async_copy(g0, acc.at[pl.ds(sid * rpw, ch)], sems0).wait()
        plsc.subcore_barrier()

        pltpu.async_copy(src_hbm.at[pl.ds(rowbase, bs)], sba, semia)
        pltpu.async_copy(dst_hbm.at[pl.ds(rowbase, bs)], dba, semia)
        pltpu.async_copy(src_hbm.at[pl.ds(rowbase + bs, bs)], sbb, semib)
        pltpu.async_copy(dst_hbm.at[pl.ds(rowbase + bs, bs)], dbb, semib)

        def run_block(sb, db, semi, bnext):
            pltpu.make_async_copy(src_hbm.at[pl.ds(rowbase, bs)], sb, semi).wait()
            pltpu.make_async_copy(dst_hbm.at[pl.ds(rowbase, bs)], db, semi).wait()

            def inner(i, carry):
                c0 = 2 * i

                @pl.when(i > 0)
                def _():
                    pltpu.make_async_copy(g0, acc.at[db.at[0]], sems0).wait()
                    pltpu.make_async_copy(g1, acc.at[db.at[0]], sems1).wait()

                pltpu.async_copy(vec_hbm.at[sb.at[c0]], g0, semg0)
                pltpu.async_copy(vec_hbm.at[sb.at[c0 + 1]], g1, semg1)
                pltpu.make_async_copy(vec_hbm.at[sb.at[c0]], g0, semg0).wait()
                pltpu.async_copy(g0, acc.at[db.at[c0]], sems0, add=True)
                pltpu.make_async_copy(vec_hbm.at[sb.at[c0]], g1, semg1).wait()
                pltpu.async_copy(g1, acc.at[db.at[c0 + 1]], sems1, add=True)
                return carry

            lax.fori_loop(0, bs // 2, inner, 0)
            pltpu.make_async_copy(g0, acc.at[db.at[0]], sems0).wait()
            pltpu.make_async_copy(g1, acc.at[db.at[0]], sems1).wait()
            pltpu.async_copy(src_hbm.at[pl.ds(rowbase + bnext * bs, bs)], sb, semi)
            pltpu.async_copy(dst_hbm.at[pl.ds(rowbase + bnext * bs, bs)], db, semi)

        def outer(bb, carry):
            b0 = 2 * bb
            run_block(sba, dba, semia, jnp.minimum(b0 + 2, nblk - 1))
            run_block(sbb, dbb, semib, jnp.minimum(b0 + 3, nblk - 1))
            return carry

        lax.fori_loop(0, nblk // 2, outer, 0)
        pltpu.make_async_copy(src_hbm.at[pl.ds(rowbase, bs)], sba, semia).wait()
        pltpu.make_async_copy(dst_hbm.at[pl.ds(rowbase, bs)], dba, semia).wait()
        pltpu.make_async_copy(src_hbm.at[pl.ds(rowbase, bs)], sbb, semib).wait()
        pltpu.make_async_copy(dst_hbm.at[pl.ds(rowbase, bs)], dbb, semib).wait()
        plsc.subcore_barrier()
        pltpu.sync_copy(acc.at[pl.ds(sid * rpw, rpw)],
                        out_hbm.at[pl.ds(cid * rows + sid * rpw, rpw)])

    return prop1_k(vec, src2d, dst2d)


def _tc_scale_matmul(x, w1, d0, d1, n, d_hid):
    def body(x_ref, w_ref, d0_ref, d1_ref, h_ref, dinv_ref):
        deg = d0_ref[...] + d1_ref[...] + 1.0
        dinv = lax.rsqrt(jnp.maximum(deg, 1.0))
        h = jnp.dot(x_ref[...], w_ref[...], preferred_element_type=jnp.float32)
        h_ref[...] = h * dinv
        dinv_ref[...] = dinv

    return pl.pallas_call(
        body,
        out_shape=(jax.ShapeDtypeStruct((n, d_hid), jnp.float32),
                   jax.ShapeDtypeStruct((n, 1), jnp.float32)),
    )(x, w1, d0, d1)


def _tc_layer2_in(p0, p1, h1s, dinv, b1, w2, n):
    def body(p0_ref, p1_ref, h_ref, dinv_ref, b1_ref, w2_ref, vs_ref):
        out1 = dinv_ref[...] * (p0_ref[...] + p1_ref[...] + h_ref[...]) + b1_ref[...]
        hrelu = jnp.maximum(out1, 0.0)
        v = jnp.dot(hrelu, w2_ref[...], preferred_element_type=jnp.float32)
        vs_ref[...] = dinv_ref[...] * v

    return pl.pallas_call(
        body,
        out_shape=jax.ShapeDtypeStruct((n, 1), jnp.float32),
    )(p0, p1, h1s, dinv, b1, w2)


def _tc_finish(t0, t1, vs, dinv, b2, n):
    def body(t0_ref, t1_ref, vs_ref, dinv_ref, b2_ref, o_ref):
        z = dinv_ref[...] * (t0_ref[...] + t1_ref[...] + vs_ref[...]) + b2_ref[...]
        o_ref[...] = 1.0 / (1.0 + jnp.exp(-z))

    return pl.pallas_call(
        body,
        out_shape=jax.ShapeDtypeStruct((n, 1), jnp.float32),
    )(t0, t1, vs, dinv, b2)


def kernel(x, edge_index, W1, b1, W2, b2):
    n, d_in = x.shape
    d_hid = W1.shape[1]
    e = edge_index.shape[1]

    nw = NC * NS
    chs, bss = 128, 8
    rpw = -(-(n + 1) // (NS * chs)) * chs
    rows = rpw * NS
    chw = 64
    nchw = -(-e // (nw * chw))
    nchw += nchw % 2
    epw = nchw * chw
    e_padw = epw * nw
    nblks = -(-e // (nw * chs * bss))
    nblks += nblks % 2
    e_pads = nw * nblks * bss * chs

    src = edge_index[0]
    dst = edge_index[1]

    def pad_edges(e_pad):
        p = e_pad - e
        psrc = (jnp.arange(p, dtype=jnp.int32)) % n
        pdst = n + (jnp.arange(p, dtype=jnp.int32)) % (rows - n)
        return (jnp.concatenate([src, psrc]), jnp.concatenate([dst, pdst]))

    src_pad, dst_pad = pad_edges(e_padw)
    srcs_f, dsts_f = pad_edges(e_pads)
    srcs = srcs_f.reshape(nw * nblks * bss, chs)
    dsts = dsts_f.reshape(nw * nblks * bss, chs)

    deg_parts = _sc_degree(dsts, rows, rpw, nblks, bss, chs)
    d0 = deg_parts[:n].reshape(n, 1)
    d1 = deg_parts[rows:rows + n].reshape(n, 1)

    h1s, dinv = _tc_scale_matmul(x, W1, d0, d1, n, d_hid)

    parts = _sc_prop_wide(h1s, src_pad, dst_pad, rows, rpw, nchw, epw, chw, d_hid)
    p0 = parts[:n]
    p1 = parts[rows:rows + n]

    vs = _tc_layer2_in(p0, p1, h1s, dinv, b1.reshape(1, d_hid), W2, n)

    t_parts = _sc_prop_scalar(vs.reshape(n), srcs, dsts, rows, rpw, nblks, bss, chs)
    t0 = t_parts[:n].reshape(n, 1)
    t1 = t_parts[rows:rows + n].reshape(n, 1)

    out = _tc_finish(t0, t1, vs, dinv, b2.reshape(1, 1), n)
    return out.reshape(n)


# speedup vs baseline: 2.8011x; 1.1147x over previous
import functools

import jax
import jax.numpy as jnp
from jax import lax
from jax.experimental import pallas as pl
from jax.experimental.pallas import tpu as pltpu
from jax.experimental.pallas import tpu_sc as plsc

NC = 2
NS = 16


def _mesh():
    return plsc.VectorSubcoreMesh(core_axis_name="c", subcore_axis_name="s")


def _sc_degree(dst2d, rows, rpw, nblk, bs, ch):
    nch = nblk * bs

    @functools.partial(
        pl.kernel,
        out_type=jax.ShapeDtypeStruct((NC * rows,), jnp.float32),
        mesh=_mesh(),
        scratch_types=[
            pltpu.VMEM((bs, ch), jnp.int32),
            pltpu.VMEM((bs, ch), jnp.int32),
            pltpu.VMEM((ch,), jnp.float32),
            pltpu.VMEM_SHARED((rows,), jnp.float32),
            pltpu.SemaphoreType.DMA,
            pltpu.SemaphoreType.DMA,
            pltpu.SemaphoreType.DMA,
            pltpu.SemaphoreType.DMA,
        ],
        name="sc_gcn_degree",
    )
    def deg_k(dst_hbm, out_hbm, dba, dbb, vals, acc, semia, semib, sems0, sems1):
        cid = lax.axis_index("c")
        sid = lax.axis_index("s")
        rowbase = (cid * NS + sid) * nch
        for k in range(ch // 16):
            vals[pl.ds(k * 16, 16)] = jnp.zeros((16,), jnp.float32)
        for r in range(rpw // ch):
            pltpu.async_copy(vals, acc.at[pl.ds(sid * rpw + r * ch, ch)], sems0)
        for r in range(rpw // ch):
            pltpu.make_async_copy(vals, acc.at[pl.ds(sid * rpw, ch)], sems0).wait()
        plsc.subcore_barrier()
        for k in range(ch // 16):
            vals[pl.ds(k * 16, 16)] = jnp.ones((16,), jnp.float32)

        pltpu.async_copy(dst_hbm.at[pl.ds(rowbase, bs)], dba, semia)
        pltpu.async_copy(dst_hbm.at[pl.ds(rowbase + bs, bs)], dbb, semib)

        def run_block(db, semi, bnext):
            pltpu.make_async_copy(dst_hbm.at[pl.ds(rowbase, bs)], db, semi).wait()

            def inner(i, carry):
                c0 = 2 * i

                @pl.when(i > 0)
                def _():
                    pltpu.make_async_copy(vals, acc.at[db.at[0]], sems0).wait()
                    pltpu.make_async_copy(vals, acc.at[db.at[0]], sems1).wait()

                pltpu.async_copy(vals, acc.at[db.at[c0]], sems0, add=True)
                pltpu.async_copy(vals, acc.at[db.at[c0 + 1]], sems1, add=True)
                return carry

            lax.fori_loop(0, bs // 2, inner, 0)
            pltpu.make_async_copy(vals, acc.at[db.at[0]], sems0).wait()
            pltpu.make_async_copy(vals, acc.at[db.at[0]], sems1).wait()
            pltpu.async_copy(dst_hbm.at[pl.ds(rowbase + bnext * bs, bs)], db, semi)

        def outer(bb, carry):
            b0 = 2 * bb
            run_block(dba, semia, jnp.minimum(b0 + 2, nblk - 1))
            run_block(dbb, semib, jnp.minimum(b0 + 3, nblk - 1))
            return carry

        lax.fori_loop(0, nblk // 2, outer, 0)
        pltpu.make_async_copy(dst_hbm.at[pl.ds(rowbase, bs)], dba, semia).wait()
        pltpu.make_async_copy(dst_hbm.at[pl.ds(rowbase, bs)], dbb, semib).wait()
        plsc.subcore_barrier()
        pltpu.sync_copy(acc.at[pl.ds(sid * rpw, rpw)],
                        out_hbm.at[pl.ds(cid * rows + sid * rpw, rpw)])

    return deg_k(dst2d)


def _sc_prop_wide(table, src_pad, dst_pad, rows, rpw, nch, epw, ch, d):
    @functools.partial(
        pl.kernel,
        out_type=jax.ShapeDtypeStruct((NC * rows, d), jnp.float32),
        mesh=_mesh(),
        scratch_types=[
            pltpu.VMEM((ch,), jnp.int32),
            pltpu.VMEM((ch,), jnp.int32),
            pltpu.VMEM((ch,), jnp.int32),
            pltpu.VMEM((ch,), jnp.int32),
            pltpu.VMEM((ch, d), jnp.float32),
            pltpu.VMEM((ch, d), jnp.float32),
            pltpu.VMEM_SHARED((rows, d), jnp.float32),
            pltpu.SemaphoreType.DMA,
            pltpu.SemaphoreType.DMA,
            pltpu.SemaphoreType.DMA,
            pltpu.SemaphoreType.DMA,
        ],
        name="sc_gcn_prop128",
    )
    def prop_k(tab_hbm, src_hbm, dst_hbm, out_hbm,
               sb0, sb1, db0, db1, g0, g1, acc,
               semi0, semi1, semg0, semg1):
        cid = lax.axis_index("c")
        sid = lax.axis_index("s")
        base = (cid * NS + sid) * epw

        def zbody(r, carry):
            for k in range(d // 16):
                g0[r, pl.ds(k * 16, 16)] = jnp.zeros((16,), jnp.float32)
            return carry

        lax.fori_loop(0, ch, zbody, 0)
        for r in range(rpw // ch):
            pltpu.sync_copy(g0, acc.at[pl.ds(sid * rpw + r * ch, ch)])
        plsc.subcore_barrier()

        pltpu.async_copy(src_hbm.at[pl.ds(base, ch)], sb0, semi0)
        pltpu.async_copy(dst_hbm.at[pl.ds(base, ch)], db0, semi0)
        pltpu.async_copy(src_hbm.at[pl.ds(base + ch, ch)], sb1, semi1)
        pltpu.async_copy(dst_hbm.at[pl.ds(base + ch, ch)], db1, semi1)
        pltpu.make_async_copy(src_hbm.at[pl.ds(base, ch)], sb0, semi0).wait()
        pltpu.make_async_copy(dst_hbm.at[pl.ds(base, ch)], db0, semi0).wait()
        pltpu.async_copy(tab_hbm.at[sb0], g0, semg0)

        def body(jj, carry):
            j = jj * 2
            o2 = base + jnp.minimum(j + 2, nch - 1) * ch
            o3 = base + jnp.minimum(j + 3, nch - 1) * ch
            pltpu.make_async_copy(src_hbm.at[pl.ds(base, ch)], sb1, semi1).wait()
            pltpu.make_async_copy(dst_hbm.at[pl.ds(base, ch)], db1, semi1).wait()
            pltpu.async_copy(tab_hbm.at[sb1], g1, semg1)
            pltpu.make_async_copy(tab_hbm.at[sb0], g0, semg0).wait()
            pltpu.sync_copy(g0, acc.at[db0], add=True)
            pltpu.async_copy(src_hbm.at[pl.ds(o2, ch)], sb0, semi0)
            pltpu.async_copy(dst_hbm.at[pl.ds(o2, ch)], db0, semi0)
            pltpu.make_async_copy(src_hbm.at[pl.ds(base, ch)], sb0, semi0).wait()
            pltpu.make_async_copy(dst_hbm.at[pl.ds(base, ch)], db0, semi0).wait()
            pltpu.async_copy(tab_hbm.at[sb0], g0, semg0)
            pltpu.make_async_copy(tab_hbm.at[sb1], g1, semg1).wait()
            pltpu.sync_copy(g1, acc.at[db1], add=True)
            pltpu.async_copy(src_hbm.at[pl.ds(o3, ch)], sb1, semi1)
            pltpu.async_copy(dst_hbm.at[pl.ds(o3, ch)], db1, semi1)
            return carry

        lax.fori_loop(0, nch // 2, body, 0)
        pltpu.make_async_copy(tab_hbm.at[sb0], g0, semg0).wait()
        pltpu.make_async_copy(src_hbm.at[pl.ds(base, ch)], sb1, semi1).wait()
        pltpu.make_async_copy(dst_hbm.at[pl.ds(base, ch)], db1, semi1).wait()
        plsc.subcore_barrier()
        pltpu.sync_copy(acc.at[pl.ds(sid * rpw, rpw)],
                        out_hbm.at[pl.ds(cid * rows + sid * rpw, rpw)])

    return prop_k(table, src_pad, dst_pad)


def _sc_prop_scalar(vec, src2d, dst2d, rows, rpw, nblk, bs, ch):
    nch = nblk * bs

    @functools.partial(
        pl.kernel,
        out_type=jax.ShapeDtypeStruct((NC * rows,), jnp.float32),
        mesh=_mesh(),
        scratch_types=[
            pltpu.VMEM((bs, ch), jnp.int32),
            pltpu.VMEM((bs, ch), jnp.int32),
            pltpu.VMEM((bs, ch), jnp.int32),
            pltpu.VMEM((bs, ch), jnp.int32),
            pltpu.VMEM((ch,), jnp.float32),
            pltpu.VMEM((ch,), jnp.float32),
            pltpu.VMEM_SHARED((rows,), jnp.float32),
            pltpu.SemaphoreType.DMA,
            pltpu.SemaphoreType.DMA,
            pltpu.SemaphoreType.DMA,
            pltpu.SemaphoreType.DMA,
            pltpu.SemaphoreType.DMA,
            pltpu.SemaphoreType.DMA,
        ],
        name="sc_gcn_prop1",
    )
    def prop1_k(vec_hbm, src_hbm, dst_hbm, out_hbm,
                sba, sbb, dba, dbb, g0, g1, acc,
                semia, semib, semg0, semg1, sems0, sems1):
        cid = lax.axis_index("c")
        sid = lax.axis_index("s")
        rowbase = (cid * NS + sid) * nch
        for k in range(ch // 16):
            g0[pl.ds(k * 16, 16)] = jnp.zeros((16,), jnp.float32)
        for r in range(rpw // ch):
            pltpu.async_copy(g0, acc.at[pl.ds(sid * rpw + r * ch, ch)], sems0)
        for r in range(rpw // ch):
            pltpu.make_async_copy(g0, acc.at[pl.ds(sid * rpw, ch)], sems0).wait()
        plsc.subcore_barrier()

        pltpu.async_copy(src_hbm.at[pl.ds(rowbase, bs)], sba, semia)
        pltpu.async_copy(dst_hbm.at[pl.ds(rowbase, bs)], dba, semia)
        pltpu.async_copy(src_hbm.at[pl.ds(rowbase + bs, bs)], sbb, semib)
        pltpu.async_copy(dst_hbm.at[pl.ds(rowbase + bs, bs)], dbb, semib)

        def run_block(sb, db, semi, bnext):
            pltpu.make_async_copy(src_hbm.at[pl.ds(rowbase, bs)], sb, semi).wait()
            pltpu.make_async_copy(dst_hbm.at[pl.ds(rowbase, bs)], db, semi).wait()

            def inner(i, carry):
                c0 = 2 * i

                @pl.when(i > 0)
                def _():
                    pltpu.make_async_copy(g0, acc.at[db.at[0]], sems0).wait()
                    pltpu.make_async_copy(g1, acc.at[db.at[0]], sems1).wait()

                pltpu.async_copy(vec_hbm.at[sb.at[c0]], g0, semg0)
                pltpu.async_copy(vec_hbm.at[sb.at[c0 + 1]], g1, semg1)
                pltpu.make_async_copy(vec_hbm.at[sb.at[c0]], g0, semg0).wait()
                pltpu.async_copy(g0, acc.at[db.at[c0]], sems0, add=True)
                pltpu.make_async_copy(vec_hbm.at[sb.at[c0]], g1, semg1).wait()
                pltpu.async_copy(g1, acc.at[db.at[c0 + 1]], sems1, add=True)
                return carry

            lax.fori_loop(0, bs // 2, inner, 0)
            pltpu.make_async_copy(g0, acc.at[db.at[0]], sems0).wait()
            pltpu.make_async_copy(g1, acc.at[db.at[0]], sems1).wait()
            pltpu.async_copy(src_hbm.at[pl.ds(rowbase + bnext * bs, bs)], sb, semi)
            pltpu.async_copy(dst_hbm.at[pl.ds(rowbase + bnext * bs, bs)], db, semi)

        def outer(bb, carry):
            b0 = 2 * bb
            run_block(sba, dba, semia, jnp.minimum(b0 + 2, nblk - 1))
            run_block(sbb, dbb, semib, jnp.minimum(b0 + 3, nblk - 1))
            return carry

        lax.fori_loop(0, nblk // 2, outer, 0)
        pltpu.make_async_copy(src_hbm.at[pl.ds(rowbase, bs)], sba, semia).wait()
        pltpu.make_async_copy(dst_hbm.at[pl.ds(rowbase, bs)], dba, semia).wait()
        pltpu.make_async_copy(src_hbm.at[pl.ds(rowbase, bs)], sbb, semib).wait()
        pltpu.make_async_copy(dst_hbm.at[pl.ds(rowbase, bs)], dbb, semib).wait()
        plsc.subcore_barrier()
        pltpu.sync_copy(acc.at[pl.ds(sid * rpw, rpw)],
                        out_hbm.at[pl.ds(cid * rows + sid * rpw, rpw)])

    return prop1_k(vec, src2d, dst2d)


def _tc_scale_matmul(x, w1, degp, n, d_hid, rows):
    def body(x_ref, w_ref, degp_ref, h_ref, dinv_ref):
        deg = degp_ref[:n] + degp_ref[rows:rows + n] + 1.0
        dinv = lax.rsqrt(jnp.maximum(deg, 1.0))
        h = jnp.dot(x_ref[...], w_ref[...], preferred_element_type=jnp.float32)
        h_ref[...] = h * dinv
        dinv_ref[...] = dinv

    return pl.pallas_call(
        body,
        out_shape=(jax.ShapeDtypeStruct((n, d_hid), jnp.float32),
                   jax.ShapeDtypeStruct((n, 1), jnp.float32)),
    )(x, w1, degp)


def _tc_layer2_in(parts, h1s, dinv, b1, w2, n, rows):
    def body(parts_ref, h_ref, dinv_ref, b1_ref, w2_ref, vs_ref):
        tmp = parts_ref[:n] + parts_ref[rows:rows + n] + h_ref[...]
        out1 = dinv_ref[...] * tmp + b1_ref[...]
        hrelu = jnp.maximum(out1, 0.0)
        v = jnp.dot(hrelu, w2_ref[...], preferred_element_type=jnp.float32)
        vs_ref[...] = dinv_ref[...] * v

    return pl.pallas_call(
        body,
        out_shape=jax.ShapeDtypeStruct((n, 1), jnp.float32),
    )(parts, h1s, dinv, b1, w2)


def _tc_finish(tp, vs, dinv, b2, n, rows):
    def body(tp_ref, vs_ref, dinv_ref, b2_ref, o_ref):
        t = tp_ref[:n] + tp_ref[rows:rows + n] + vs_ref[...]
        z = dinv_ref[...] * t + b2_ref[...]
        o_ref[...] = 1.0 / (1.0 + jnp.exp(-z))

    return pl.pallas_call(
        body,
        out_shape=jax.ShapeDtypeStruct((n, 1), jnp.float32),
    )(tp, vs, dinv, b2)


def kernel(x, edge_index, W1, b1, W2, b2):
    n, d_in = x.shape
    d_hid = W1.shape[1]
    e = edge_index.shape[1]

    nw = NC * NS
    chs, bss = 128, 8
    rpw = -(-(n + 1) // (NS * chs)) * chs
    rows = rpw * NS
    chw = 80
    nchw = -(-e // (nw * chw))
    nchw += nchw % 2
    epw = nchw * chw
    e_padw = epw * nw
    nblks = -(-e // (nw * chs * bss))
    nblks += nblks % 2
    e_pads = nw * nblks * bss * chs

    src = edge_index[0]
    dst = edge_index[1]

    def pad_edges(e_pad):
        p = e_pad - e
        psrc = (jnp.arange(p, dtype=jnp.int32)) % n
        pdst = n + (jnp.arange(p, dtype=jnp.int32)) % (rows - n)
        return (jnp.concatenate([src, psrc]), jnp.concatenate([dst, pdst]))

    src_pad, dst_pad = pad_edges(e_padw)
    srcs_f, dsts_f = pad_edges(e_pads)
    srcs = srcs_f.reshape(nw * nblks * bss, chs)
    dsts = dsts_f.reshape(nw * nblks * bss, chs)

    deg_parts = _sc_degree(dsts, rows, rpw, nblks, bss, chs)

    h1s, dinv = _tc_scale_matmul(x, W1, deg_parts.reshape(NC * rows, 1),
                                 n, d_hid, rows)

    parts = _sc_prop_wide(h1s, src_pad, dst_pad, rows, rpw, nchw, epw, chw, d_hid)

    vs = _tc_layer2_in(parts, h1s, dinv, b1.reshape(1, d_hid), W2, n, rows)

    t_parts = _sc_prop_scalar(vs.reshape(n), srcs, dsts, rows, rpw, nblks, bss, chs)

    out = _tc_finish(t_parts.reshape(NC * rows, 1), vs, dinv,
                     b2.reshape(1, 1), n, rows)
    return out.reshape(n)
